# Initial kernel scaffold; baseline (speedup 1.0000x reference)
import jax
import jax.numpy as jnp
from jax.experimental import pallas as pl


def kernel(tokens, emb):
    raise NotImplementedError("write your pallas kernel here")


# speedup vs baseline: 1.5007x; 1.5007x over previous
import jax
import jax.numpy as jnp
from jax.experimental import pallas as pl
from jax.experimental.pallas import tpu as pltpu

_BN = 512


def _add_body(tok_ref, emb_ref, out_ref):
    out_ref[...] = tok_ref[...] + emb_ref[...]


def kernel(tokens, emb):
    b, n, c = tokens.shape
    grid = (n // _BN, b)
    return pl.pallas_call(
        _add_body,
        grid=grid,
        in_specs=[
            pl.BlockSpec((1, _BN, c), lambda i, j: (j, i, 0)),
            pl.BlockSpec((_BN, c), lambda i, j: (i, 0)),
        ],
        out_specs=pl.BlockSpec((1, _BN, c), lambda i, j: (j, i, 0)),
        out_shape=jax.ShapeDtypeStruct((b, n, c), tokens.dtype),
    )(tokens, emb)


# speedup vs baseline: 1.6657x; 1.1099x over previous
import jax
import jax.numpy as jnp
from jax.experimental import pallas as pl
from jax.experimental.pallas import tpu as pltpu

_BN = 1024


def _add_body(tok_ref, emb_ref, out_ref):
    out_ref[...] = tok_ref[...] + emb_ref[...]


def kernel(tokens, emb):
    b, n, c = tokens.shape
    grid = (n // _BN, b)
    return pl.pallas_call(
        _add_body,
        grid=grid,
        in_specs=[
            pl.BlockSpec((1, _BN, c), lambda i, j: (j, i, 0)),
            pl.BlockSpec((_BN, c), lambda i, j: (i, 0)),
        ],
        out_specs=pl.BlockSpec((1, _BN, c), lambda i, j: (j, i, 0)),
        out_shape=jax.ShapeDtypeStruct((b, n, c), tokens.dtype),
    )(tokens, emb)


# speedup vs baseline: 1.7400x; 1.0446x over previous
import jax
import jax.numpy as jnp
from jax.experimental import pallas as pl
from jax.experimental.pallas import tpu as pltpu

_BN = 2048


def _add_body(tok_ref, emb_ref, out_ref):
    out_ref[...] = tok_ref[...] + emb_ref[...]


def kernel(tokens, emb):
    b, n, c = tokens.shape
    grid = (n // _BN, b)
    return pl.pallas_call(
        _add_body,
        grid=grid,
        in_specs=[
            pl.BlockSpec((1, _BN, c), lambda i, j: (j, i, 0)),
            pl.BlockSpec((_BN, c), lambda i, j: (i, 0)),
        ],
        out_specs=pl.BlockSpec((1, _BN, c), lambda i, j: (j, i, 0)),
        out_shape=jax.ShapeDtypeStruct((b, n, c), tokens.dtype),
    )(tokens, emb)
